# baseline (speedup 1.0000x reference)
import functools

import jax
import jax.numpy as jnp
from jax import lax
from jax.experimental import pallas as pl
from jax.experimental.pallas import tpu as pltpu
from jax.experimental.pallas import tpu_sc as plsc


def kernel(Y, X, x_idx, y_idx):
    Z, H, W = Y.shape
    n = X.shape[1]
    N = Z * H * W
    info = plsc.get_sparse_core_info()
    NC, NS = info.num_cores, info.num_subcores
    NW = NC * NS
    chunk = N // NW
    npairs = (Z // NW) * n
    rows = npairs // 128

    Yf = Y.reshape(N)
    Xf = X.reshape(Z * n)
    xi = x_idx.astype(jnp.int32)
    yi = y_idx.astype(jnp.int32)

    mesh = plsc.VectorSubcoreMesh(core_axis_name="c", subcore_axis_name="s")

    @functools.partial(
        pl.kernel,
        mesh=mesh,
        out_type=jax.ShapeDtypeStruct((N,), jnp.float32),
        scratch_types=[
            pltpu.VMEM((n,), jnp.int32),
            pltpu.VMEM((n,), jnp.int32),
            pltpu.VMEM((rows, 128), jnp.int32),
            pltpu.VMEM((rows, 128), jnp.float32),
            pltpu.VMEM((npairs,), jnp.float32),
            pltpu.SemaphoreType.DMA,
            pltpu.SemaphoreType.DMA,
        ],
    )
    def sc_run(y_hbm, x_hbm, xi_hbm, yi_hbm, out_hbm,
               xi_v, yi_v, idx_v, vals_v, xv_v, csem, gsem):
        c = lax.axis_index("c")
        s = lax.axis_index("s")
        w = s * NC + c
        base = w * chunk
        big = pltpu.async_copy(y_hbm.at[pl.ds(base, chunk)],
                               out_hbm.at[pl.ds(base, chunk)], csem)

        pltpu.sync_copy(xi_hbm, xi_v)
        pltpu.sync_copy(yi_hbm, yi_v)
        pltpu.sync_copy(x_hbm.at[pl.ds(w * npairs, npairs)], xv_v)

        pbase = w * npairs
        period = n // 16
        for r in range(rows):
            for k in range(128 // 16):
                v = r * (128 // 16) + k
                m = v % period
                zs = lax.shift_right_logical(pbase + v * 16, 6)
                xg = xi_v[pl.ds(m * 16, 16)]
                yg = yi_v[pl.ds(m * 16, 16)]
                idx_v[r, pl.ds(k * 16, 16)] = zs * (H * W) + xg * W + yg

        for r in range(rows):
            pltpu.async_copy(y_hbm.at[idx_v.at[r]], vals_v.at[r], gsem).wait()

        for r in range(rows):
            for k in range(128 // 16):
                sl = pl.ds(k * 16, 16)
                vals_v[r, sl] = vals_v[r, sl] + xv_v[pl.ds(r * 128 + k * 16, 16)]

        big.wait()
        for r in range(rows):
            pltpu.sync_copy(vals_v.at[r], out_hbm.at[idx_v.at[r]])

    out = sc_run(Yf, Xf, xi, yi)
    return out.reshape(Z, H, W)


# speedup vs baseline: 12.9106x; 12.9106x over previous
import functools

import jax
import jax.numpy as jnp
from jax import lax
from jax.experimental import pallas as pl
from jax.experimental.pallas import tpu as pltpu
from jax.experimental.pallas import tpu_sc as plsc


_CH = 32768


def kernel(Y, X, x_idx, y_idx):
    Z, H, W = Y.shape
    n = X.shape[1]
    N = Z * H * W
    info = plsc.get_sparse_core_info()
    NC, NS = info.num_cores, info.num_subcores
    NW = NC * NS
    chunk = N // NW
    zpw = Z // NW
    slice_words = H * W
    cps = slice_words // _CH
    nch = chunk // _CH
    nxv = zpw * n
    ngrp = n // 16

    Yf = Y.reshape(N)
    Xf = X.reshape(Z * n)
    xi = x_idx.astype(jnp.int32)
    yi = y_idx.astype(jnp.int32)

    mesh = plsc.VectorSubcoreMesh(core_axis_name="c", subcore_axis_name="s")

    @functools.partial(
        pl.kernel,
        mesh=mesh,
        out_type=jax.ShapeDtypeStruct((N,), jnp.float32),
        scratch_types=[
            pltpu.VMEM((n,), jnp.int32),
            pltpu.VMEM((n,), jnp.int32),
            pltpu.VMEM((nxv,), jnp.float32),
            pltpu.VMEM((_CH,), jnp.float32),
            pltpu.VMEM((_CH,), jnp.float32),
            pltpu.SemaphoreType.DMA,
            pltpu.SemaphoreType.DMA,
            pltpu.SemaphoreType.DMA,
            pltpu.SemaphoreType.DMA,
        ],
        compiler_params=pltpu.CompilerParams(needs_layout_passes=False),
    )
    def sc_run(y_hbm, x_hbm, xi_hbm, yi_hbm, out_hbm,
               xi_v, yi_v, xv_v, buf0, buf1, g0, g1, s0, s1):
        c_ax = lax.axis_index("c")
        s_ax = lax.axis_index("s")
        w = s_ax * NC + c_ax
        base = w * chunk
        gsem = (g0, g1)
        ssem = (s0, s1)
        bufs = (buf0, buf1)

        pltpu.sync_copy(xi_hbm, xi_v)
        pltpu.sync_copy(yi_hbm, yi_v)
        pltpu.sync_copy(x_hbm.at[pl.ds(w * nxv, nxv)], xv_v)

        offs = [xi_v[pl.ds(t * 16, 16)] * W + yi_v[pl.ds(t * 16, 16)]
                for t in range(ngrp)]

        def start_gather(c):
            b = c & 1
            return pltpu.async_copy(
                y_hbm.at[pl.ds(base + c * _CH, _CH)], bufs[b], gsem[b])

        def start_scatter(c):
            b = c & 1
            return pltpu.async_copy(
                bufs[b], out_hbm.at[pl.ds(base + c * _CH, _CH)], ssem[b])

        g = {0: start_gather(0)}
        s = {}
        for c in range(nch):
            b = c & 1
            if c >= 1:
                s[c - 1].wait()
            if c + 1 < nch:
                g[c + 1] = start_gather(c + 1)
            g[c].wait()
            zrel, sub = c // cps, c % cps
            for t in range(ngrp):
                rel = offs[t] - sub * _CH
                mask = jnp.logical_and(rel >= 0, rel < _CH)
                vals = xv_v[pl.ds((zrel * ngrp + t) * 16, 16)]
                plsc.addupdate_scatter(bufs[b], [rel], vals, mask=mask)
            s[c] = start_scatter(c)
        s[nch - 1].wait()

    out = sc_run(Yf, Xf, xi, yi)
    return out.reshape(Z, H, W)


# speedup vs baseline: 13.0334x; 1.0095x over previous
import functools

import jax
import jax.numpy as jnp
from jax import lax
from jax.experimental import pallas as pl
from jax.experimental.pallas import tpu as pltpu
from jax.experimental.pallas import tpu_sc as plsc


_BS = 8


def _copy_body(y_ref, out_ref):
    out_ref[...] = y_ref[...]


def _tc_copy(Yf, blk):
    N = Yf.shape[0]
    return pl.pallas_call(
        _copy_body,
        grid=(N // blk,),
        in_specs=[pl.BlockSpec((blk,), lambda i: (i,))],
        out_specs=pl.BlockSpec((blk,), lambda i: (i,)),
        out_shape=jax.ShapeDtypeStruct((N,), jnp.float32),
    )(Yf)


def kernel(Y, X, x_idx, y_idx):
    Z, H, W = Y.shape
    n = X.shape[1]
    N = Z * H * W
    info = plsc.get_sparse_core_info()
    NC, NS = info.num_cores, info.num_subcores
    NW = NC * NS
    npairs = (Z // NW) * n
    rows = npairs // 128
    ngrp = 128 // 16

    Yf = Y.reshape(N)
    Xf = X.reshape(Z * n)
    xi = x_idx.astype(jnp.int32)
    yi = y_idx.astype(jnp.int32)

    mesh = plsc.VectorSubcoreMesh(core_axis_name="c", subcore_axis_name="s")

    @functools.partial(
        pl.kernel,
        mesh=mesh,
        out_type=(),
        scratch_types=[
            pltpu.VMEM((n,), jnp.int32),
            pltpu.VMEM((n,), jnp.int32),
            pltpu.VMEM((rows, 128), jnp.int32),
            pltpu.VMEM((rows, 128), jnp.float32),
            pltpu.VMEM((npairs,), jnp.float32),
            pltpu.SemaphoreType.DMA,
        ],
        compiler_params=pltpu.CompilerParams(needs_layout_passes=False),
    )
    def sc_inject(out_hbm, x_hbm, xi_hbm, yi_hbm,
                  xi_v, yi_v, idx_v, vals_v, xv_v, gsem):
        c_ax = lax.axis_index("c")
        s_ax = lax.axis_index("s")
        w = s_ax * NC + c_ax

        pltpu.sync_copy(xi_hbm, xi_v)
        pltpu.sync_copy(yi_hbm, yi_v)
        pltpu.sync_copy(x_hbm.at[pl.ds(w * npairs, npairs)], xv_v)

        pbase = w * npairs
        period = n // 16
        for r in range(rows):
            for k in range(ngrp):
                v = r * ngrp + k
                m = v % period
                zs = lax.shift_right_logical(pbase + v * 16, 6)
                xg = xi_v[pl.ds(m * 16, 16)]
                yg = yi_v[pl.ds(m * 16, 16)]
                idx_v[r, pl.ds(k * 16, 16)] = zs * (H * W) + xg * W + yg

        for r in range(rows):
            pltpu.async_copy(out_hbm.at[idx_v.at[r]], vals_v.at[r], gsem).wait()

        for r in range(rows):
            for k in range(ngrp):
                sl = pl.ds(k * 16, 16)
                vals_v[r, sl] = vals_v[r, sl] + xv_v[pl.ds(r * 128 + k * 16, 16)]

        for r in range(rows):
            pltpu.sync_copy(vals_v.at[r], out_hbm.at[idx_v.at[r]])

    out_ref = jax.new_ref(_tc_copy(Yf, _BS * H * W))
    sc_inject(out_ref, Xf, xi, yi)
    return jax.freeze(out_ref).reshape(Z, H, W)


# speedup vs baseline: 50.5903x; 3.8816x over previous
import jax
import jax.numpy as jnp
from jax.experimental import pallas as pl
from jax.experimental.pallas import tpu as pltpu


_BS = 8


def _inject_body(xv_ref, yv_ref, y_ref, x_ref, out_ref, a_ref, m_ref):
    H, n = y_ref.shape[1], xv_ref.shape[2]

    @pl.when(pl.program_id(0) == 0)
    def _build_onehots():
        riota = jax.lax.broadcasted_iota(jnp.int32, (H, n), 0)
        a_ref[...] = (riota == xv_ref[0]).astype(jnp.bfloat16)
        m_ref[...] = (riota == yv_ref[0]).astype(jnp.bfloat16)

    A = a_ref[...]
    M = m_ref[...]
    for b in range(y_ref.shape[0]):
        scaled = A * x_ref[b]
        D = jax.lax.dot_general(
            scaled, M, (((1,), (1,)), ((), ())),
            preferred_element_type=jnp.float32)
        out_ref[b] = y_ref[b] + D


def kernel(Y, X, x_idx, y_idx):
    Z, H, W = Y.shape
    n = X.shape[1]
    xv = x_idx.astype(jnp.int32).reshape(1, 1, n)
    yv = y_idx.astype(jnp.int32).reshape(1, 1, n)
    X3 = X.reshape(Z, 1, n).astype(jnp.bfloat16)
    grid = (Z // _BS,)
    out = pl.pallas_call(
        _inject_body,
        grid=grid,
        in_specs=[
            pl.BlockSpec((1, 1, n), lambda z: (0, 0, 0)),
            pl.BlockSpec((1, 1, n), lambda z: (0, 0, 0)),
            pl.BlockSpec((_BS, H, W), lambda z: (z, 0, 0)),
            pl.BlockSpec((_BS, 1, n), lambda z: (z, 0, 0)),
        ],
        out_specs=pl.BlockSpec((_BS, H, W), lambda z: (z, 0, 0)),
        out_shape=jax.ShapeDtypeStruct((Z, H, W), jnp.float32),
        scratch_shapes=[
            pltpu.VMEM((H, n), jnp.bfloat16),
            pltpu.VMEM((H, n), jnp.bfloat16),
        ],
    )(xv, yv, Y, X3)
    return out


# speedup vs baseline: 50.7699x; 1.0036x over previous
import jax
import jax.numpy as jnp
from jax.experimental import pallas as pl
from jax.experimental.pallas import tpu as pltpu


_BS = 8


def _inject_body(xv_ref, yv_ref, y_ref, x_ref, out_ref, a_ref, m_ref):
    H, n = y_ref.shape[1], xv_ref.shape[2]

    @pl.when(pl.program_id(0) == 0)
    def _build_onehots():
        riota = jax.lax.broadcasted_iota(jnp.int32, (H, n), 0)
        a_ref[...] = (riota == xv_ref[0]).astype(jnp.float32)
        m_ref[...] = (riota == yv_ref[0]).astype(jnp.float32)

    A = a_ref[...]
    M = m_ref[...]
    for b in range(y_ref.shape[0]):
        scaled = A * x_ref[b]
        D = jax.lax.dot_general(
            scaled, M, (((1,), (1,)), ((), ())),
            preferred_element_type=jnp.float32)
        out_ref[b] = y_ref[b] + D


def kernel(Y, X, x_idx, y_idx):
    Z, H, W = Y.shape
    n = X.shape[1]
    xv = x_idx.astype(jnp.int32).reshape(1, 1, n)
    yv = y_idx.astype(jnp.int32).reshape(1, 1, n)
    X3 = X.reshape(Z, 1, n)
    grid = (Z // _BS,)
    out = pl.pallas_call(
        _inject_body,
        grid=grid,
        in_specs=[
            pl.BlockSpec((1, 1, n), lambda z: (0, 0, 0)),
            pl.BlockSpec((1, 1, n), lambda z: (0, 0, 0)),
            pl.BlockSpec((_BS, H, W), lambda z: (z, 0, 0)),
            pl.BlockSpec((_BS, 1, n), lambda z: (z, 0, 0)),
        ],
        out_specs=pl.BlockSpec((_BS, H, W), lambda z: (z, 0, 0)),
        out_shape=jax.ShapeDtypeStruct((Z, H, W), jnp.float32),
        scratch_shapes=[
            pltpu.VMEM((H, n), jnp.float32),
            pltpu.VMEM((H, n), jnp.float32),
        ],
    )(xv, yv, Y, X3)
    return out
